# Initial kernel scaffold; baseline (speedup 1.0000x reference)
import functools

import jax
import jax.numpy as jnp
from jax import lax
from jax.experimental import pallas as pl
from jax.experimental.pallas import tpu as pltpu
from jax.experimental.pallas import tpu_sc as plsc

VOCAB_SZ = 100000
TABLE_SZ = 1000000
NUM_ALT = 4
NTOK = 16384
PSEUDO_CT = 5.0

NWORKERS = 32
CHUNK = NTOK // NWORKERS
NVEC = CHUNK // 16
FLAT = NUM_ALT * CHUNK

_LN2 = 0.6931471805599453
_SQRT2 = 1.4142135623730951


def _ln(x):
    bits = lax.bitcast_convert_type(x, jnp.int32)
    e = lax.shift_right_arithmetic(bits, 23) - 127
    m = lax.bitcast_convert_type(
        (bits & 0x007FFFFF) | 0x3F800000, jnp.float32)
    big = m > _SQRT2
    m = jnp.where(big, m * 0.5, m)
    e = jnp.where(big, e + 1, e)
    s = (m - 1.0) / (m + 1.0)
    z = s * s
    poly = s * (2.0 + z * (2.0 / 3.0 + z * (2.0 / 5.0 + z * (2.0 / 7.0))))
    return e.astype(jnp.float32) * _LN2 + poly


_MESH = plsc.VectorSubcoreMesh(core_axis_name="c", subcore_axis_name="s")


@functools.partial(
    pl.kernel,
    mesh=_MESH,
    out_type=jax.ShapeDtypeStruct((NTOK, NUM_ALT), jnp.float32),
    scratch_types=[
        pltpu.VMEM((CHUNK,), jnp.int32),
        pltpu.VMEM((CHUNK,), jnp.int32),
        pltpu.VMEM((CHUNK,), jnp.int32),
        pltpu.VMEM((CHUNK, NUM_ALT), jnp.int32),
        pltpu.VMEM((FLAT,), jnp.int32),
        pltpu.VMEM((FLAT,), jnp.int32),
        pltpu.VMEM((FLAT,), jnp.int32),
        pltpu.VMEM((FLAT,), jnp.int32),
        pltpu.VMEM((FLAT,), jnp.float32),
        pltpu.VMEM((FLAT,), jnp.float32),
        pltpu.VMEM((FLAT,), jnp.float32),
        pltpu.VMEM((FLAT,), jnp.float32),
        pltpu.VMEM((CHUNK, NUM_ALT), jnp.float32),
        pltpu.SemaphoreType.DMA,
    ],
)
def _score_kernel(tok_hbm, prev_hbm, nxt_hbm, casing_hbm,
                  uni_hbm, bb_hbm, bf_hbm, tri_hbm, out_hbm,
                  tok_v, prev_v, nxt_v, alts_v,
                  uidx_v, bbidx_v, bfidx_v, tridx_v,
                  uval_v, bbval_v, bfval_v, trval_v,
                  outb_v, sem):
    wid = lax.axis_index("s") * 2 + lax.axis_index("c")
    base = wid * CHUNK

    pltpu.sync_copy(tok_hbm.at[pl.ds(base, CHUNK)], tok_v)
    pltpu.sync_copy(prev_hbm.at[pl.ds(base, CHUNK)], prev_v)
    pltpu.sync_copy(nxt_hbm.at[pl.ds(base, CHUNK)], nxt_v)
    pltpu.async_copy(casing_hbm.at[tok_v], alts_v, sem).wait()

    lane = lax.iota(jnp.int32, 16)

    def hash_body(i, carry):
        off = i * 16
        row = off + lane
        p = prev_v[pl.ds(off, 16)]
        nx = nxt_v[pl.ds(off, 16)]
        for a in range(NUM_ALT):
            acol = jnp.full((16,), a, jnp.int32)
            alt = plsc.load_gather(alts_v, [row, acol])
            fo = a * CHUNK + off
            uidx_v[pl.ds(fo, 16)] = alt
            bbidx_v[pl.ds(fo, 16)] = (
                jnp.abs(p * 1000003 + alt * 97) % TABLE_SZ)
            bfidx_v[pl.ds(fo, 16)] = (
                jnp.abs(alt * 1000003 + nx * 97) % TABLE_SZ)
            tridx_v[pl.ds(fo, 16)] = (
                jnp.abs(p * 1000003 + alt * 97 + nx * 31337) % TABLE_SZ)
        return carry

    lax.fori_loop(0, NVEC, hash_body, 0)

    copies = [
        pltpu.async_copy(uni_hbm.at[uidx_v], uval_v, sem),
        pltpu.async_copy(bb_hbm.at[bbidx_v], bbval_v, sem),
        pltpu.async_copy(bf_hbm.at[bfidx_v], bfval_v, sem),
        pltpu.async_copy(tri_hbm.at[tridx_v], trval_v, sem),
    ]
    for c in copies:
        c.wait()

    def score_body(i, carry):
        off = i * 16
        row = off + lane
        u, b1, b2, t = [], [], [], []
        for a in range(NUM_ALT):
            fo = a * CHUNK + off
            u.append(uval_v[pl.ds(fo, 16)] + PSEUDO_CT)
            b1.append(bbval_v[pl.ds(fo, 16)] + PSEUDO_CT)
            b2.append(bfval_v[pl.ds(fo, 16)] + PSEUDO_CT)
            t.append(trval_v[pl.ds(fo, 16)] + PSEUDO_CT)
        su = (u[0] + u[1]) + (u[2] + u[3])
        sb1 = (b1[0] + b1[1]) + (b1[2] + b1[3])
        sb2 = (b2[0] + b2[1]) + (b2[2] + b2[3])
        st = (t[0] + t[1]) + (t[2] + t[3])
        den = _ln((su * sb1) * (sb2 * st))
        for a in range(NUM_ALT):
            acol = jnp.full((16,), a, jnp.int32)
            num = _ln((u[a] * b1[a]) * (b2[a] * t[a]))
            plsc.store_scatter(outb_v, [row, acol], num - den)
        return carry

    lax.fori_loop(0, NVEC, score_body, 0)

    pltpu.sync_copy(outb_v, out_hbm.at[pl.ds(base, CHUNK)])


def kernel(tokens, casing_lookup, uni_counts, bi_back_counts,
           bi_fwd_counts, tri_counts):
    prev = jnp.concatenate([tokens[:1], tokens[:-1]])
    nxt = jnp.concatenate([tokens[1:], tokens[-1:]])
    return _score_kernel(tokens, prev, nxt, casing_lookup,
                         uni_counts, bi_back_counts, bi_fwd_counts,
                         tri_counts)


# speedup vs baseline: 1.6386x; 1.6386x over previous
import functools

import jax
import jax.numpy as jnp
from jax import lax
from jax.experimental import pallas as pl
from jax.experimental.pallas import tpu as pltpu
from jax.experimental.pallas import tpu_sc as plsc

VOCAB_SZ = 100000
TABLE_SZ = 1000000
NUM_ALT = 4
NTOK = 16384
PSEUDO_CT = 5.0

NWORKERS = 32
CHUNK = NTOK // NWORKERS
NVEC = CHUNK // 16
FLAT = NUM_ALT * CHUNK

_LN2 = 0.6931471805599453
_SQRT2 = 1.4142135623730951


def _ln(x):
    bits = lax.bitcast_convert_type(x, jnp.int32)
    e = lax.shift_right_arithmetic(bits, 23) - 127
    m = lax.bitcast_convert_type(
        (bits & 0x007FFFFF) | 0x3F800000, jnp.float32)
    big = m > _SQRT2
    m = jnp.where(big, m * 0.5, m)
    e = jnp.where(big, e + 1, e)
    s = (m - 1.0) / (m + 1.0)
    z = s * s
    poly = s * (2.0 + z * (2.0 / 3.0 + z * (2.0 / 5.0 + z * (2.0 / 7.0))))
    return e.astype(jnp.float32) * _LN2 + poly


_MESH = plsc.VectorSubcoreMesh(core_axis_name="c", subcore_axis_name="s")


@functools.partial(
    pl.kernel,
    mesh=_MESH,
    out_type=jax.ShapeDtypeStruct((NUM_ALT, NTOK), jnp.float32),
    scratch_types=[
        pltpu.VMEM((CHUNK,), jnp.int32),
        pltpu.VMEM((CHUNK,), jnp.int32),
        pltpu.VMEM((CHUNK,), jnp.int32),
        pltpu.VMEM((FLAT,), jnp.int32),
        pltpu.VMEM((FLAT,), jnp.int32),
        pltpu.VMEM((FLAT,), jnp.int32),
        pltpu.VMEM((FLAT,), jnp.int32),
        pltpu.VMEM((FLAT,), jnp.int32),
        pltpu.VMEM((FLAT,), jnp.float32),
        pltpu.VMEM((FLAT,), jnp.float32),
        pltpu.VMEM((FLAT,), jnp.float32),
        pltpu.VMEM((FLAT,), jnp.float32),
        pltpu.VMEM((FLAT,), jnp.float32),
        pltpu.SemaphoreType.DMA,
    ],
)
def _score_kernel(tok_hbm, prev_hbm, nxt_hbm, casing_hbm,
                  uni_hbm, bb_hbm, bf_hbm, tri_hbm, out_hbm,
                  tok_v, prev_v, nxt_v, cidx_v,
                  uidx_v, bbidx_v, bfidx_v, tridx_v,
                  uval_v, bbval_v, bfval_v, trval_v,
                  outb_v, sem):
    wid = lax.axis_index("s") * 2 + lax.axis_index("c")
    base = wid * CHUNK

    pltpu.sync_copy(tok_hbm.at[pl.ds(base, CHUNK)], tok_v)
    pltpu.sync_copy(prev_hbm.at[pl.ds(base, CHUNK)], prev_v)
    pltpu.sync_copy(nxt_hbm.at[pl.ds(base, CHUNK)], nxt_v)

    def cidx_body(i, carry):
        off = i * 16
        t4 = tok_v[pl.ds(off, 16)] * NUM_ALT
        for a in range(NUM_ALT):
            cidx_v[pl.ds(a * CHUNK + off, 16)] = t4 + a
        return carry

    lax.fori_loop(0, NVEC, cidx_body, 0)
    pltpu.async_copy(casing_hbm.at[cidx_v], uidx_v, sem).wait()

    def hash_body(i, carry):
        off = i * 16
        p = prev_v[pl.ds(off, 16)]
        nx = nxt_v[pl.ds(off, 16)]
        for a in range(NUM_ALT):
            fo = a * CHUNK + off
            alt = uidx_v[pl.ds(fo, 16)]
            bbidx_v[pl.ds(fo, 16)] = (
                jnp.abs(p * 1000003 + alt * 97) % TABLE_SZ)
            bfidx_v[pl.ds(fo, 16)] = (
                jnp.abs(alt * 1000003 + nx * 97) % TABLE_SZ)
            tridx_v[pl.ds(fo, 16)] = (
                jnp.abs(p * 1000003 + alt * 97 + nx * 31337) % TABLE_SZ)
        return carry

    lax.fori_loop(0, NVEC, hash_body, 0)

    copies = [
        pltpu.async_copy(uni_hbm.at[uidx_v], uval_v, sem),
        pltpu.async_copy(bb_hbm.at[bbidx_v], bbval_v, sem),
        pltpu.async_copy(bf_hbm.at[bfidx_v], bfval_v, sem),
        pltpu.async_copy(tri_hbm.at[tridx_v], trval_v, sem),
    ]
    for c in copies:
        c.wait()

    def score_body(i, carry):
        off = i * 16
        u, b1, b2, t = [], [], [], []
        for a in range(NUM_ALT):
            fo = a * CHUNK + off
            u.append(uval_v[pl.ds(fo, 16)] + PSEUDO_CT)
            b1.append(bbval_v[pl.ds(fo, 16)] + PSEUDO_CT)
            b2.append(bfval_v[pl.ds(fo, 16)] + PSEUDO_CT)
            t.append(trval_v[pl.ds(fo, 16)] + PSEUDO_CT)
        su = (u[0] + u[1]) + (u[2] + u[3])
        sb1 = (b1[0] + b1[1]) + (b1[2] + b1[3])
        sb2 = (b2[0] + b2[1]) + (b2[2] + b2[3])
        st = (t[0] + t[1]) + (t[2] + t[3])
        den = _ln((su * sb1) * (sb2 * st))
        for a in range(NUM_ALT):
            num = _ln((u[a] * b1[a]) * (b2[a] * t[a]))
            outb_v[pl.ds(a * CHUNK + off, 16)] = num - den
        return carry

    lax.fori_loop(0, NVEC, score_body, 0)

    for a in range(NUM_ALT):
        pltpu.sync_copy(outb_v.at[pl.ds(a * CHUNK, CHUNK)],
                        out_hbm.at[a, pl.ds(base, CHUNK)])


def kernel(tokens, casing_lookup, uni_counts, bi_back_counts,
           bi_fwd_counts, tri_counts):
    prev = jnp.concatenate([tokens[:1], tokens[:-1]])
    nxt = jnp.concatenate([tokens[1:], tokens[-1:]])
    out = _score_kernel(tokens, prev, nxt,
                        casing_lookup.reshape(-1),
                        uni_counts, bi_back_counts, bi_fwd_counts,
                        tri_counts)
    return out.T
